# baseline (speedup 1.0000x reference)
import functools

import jax
import jax.numpy as jnp
from jax import lax
from jax.experimental import pallas as pl
from jax.experimental.pallas import tpu as pltpu
from jax.experimental.pallas import tpu_sc as plsc

_VOCAB = 1000
_N1 = 1024
_N2 = 1024
_LANES = 16
_NC = 2
_NS = 16
_NW = _NC * _NS
_PER_W = _N1 // _NW


def _sc_diag_gather_body(seq1_hbm, pw_flat_hbm, out_hbm, idx_raw, idx_sc,
                         vals, sem):
    wid = lax.axis_index("s") * _NC + lax.axis_index("c")
    base = wid * _PER_W
    pltpu.sync_copy(seq1_hbm.at[pl.ds(base, _PER_W)], idx_raw)
    for j in range(_PER_W // _LANES):
        v = idx_raw[pl.ds(j * _LANES, _LANES)]
        idx_sc[pl.ds(j * _LANES, _LANES)] = v * (_VOCAB + 1)
    pltpu.async_copy(pw_flat_hbm.at[idx_sc], vals, sem).wait()
    pltpu.sync_copy(vals, out_hbm.at[pl.ds(base, _PER_W)])


_sc_diag_gather = functools.partial(
    pl.kernel,
    out_type=jax.ShapeDtypeStruct((_N1, 1), jnp.float32),
    mesh=plsc.VectorSubcoreMesh(core_axis_name="c", subcore_axis_name="s"),
    scratch_types=[
        pltpu.VMEM((_PER_W,), jnp.int32),
        pltpu.VMEM((_PER_W,), jnp.int32),
        pltpu.VMEM((_PER_W, 1), jnp.float32),
        pltpu.SemaphoreType.DMA,
    ],
    compiler_params=pltpu.CompilerParams(use_tc_tiling_on_sc=False),
)(_sc_diag_gather_body)


def _fill_body(s1_ref, d1_ref, s2e_ref, base_ref, out_ref):
    eq = s1_ref[...] == s2e_ref[...]
    out_ref[...] = jnp.where(eq, d1_ref[...], base_ref[...])


def kernel(encoded_seq1, encoded_seq2, pw_scores, gap_score):
    n1, n2 = _N1, _N2
    w = 3 * (n2 + 1)
    gap = gap_score.astype(jnp.float32)

    pw_flat = pw_scores.reshape(_VOCAB * _VOCAB, 1)
    d1 = _sc_diag_gather(encoded_seq1, pw_flat)

    lane = jnp.arange(w, dtype=jnp.int32)
    ch0 = lane % 3 == 0
    s2rep = jnp.repeat(encoded_seq2, 3, total_repeat_length=3 * n2)
    s2rep = jnp.concatenate(
        [s2rep, jnp.full((3,), -1, dtype=jnp.int32)])
    s2e = jnp.where(ch0, s2rep, -1).reshape(1, w)
    base = jnp.where(ch0, jnp.float32(0.0), gap).reshape(1, w)
    s1p = jnp.concatenate(
        [encoded_seq1, jnp.full((1,), -2, dtype=jnp.int32)]).reshape(n1 + 1, 1)
    d1p = jnp.concatenate(
        [d1, jnp.zeros((1, 1), dtype=jnp.float32)], axis=0)

    bi = 128
    grid = (pl.cdiv(n1 + 1, bi),)
    out_flat = pl.pallas_call(
        _fill_body,
        grid=grid,
        in_specs=[
            pl.BlockSpec((bi, 1), lambda i: (i, 0)),
            pl.BlockSpec((bi, 1), lambda i: (i, 0)),
            pl.BlockSpec((1, w), lambda i: (0, 0)),
            pl.BlockSpec((1, w), lambda i: (0, 0)),
        ],
        out_specs=pl.BlockSpec((bi, w), lambda i: (i, 0)),
        out_shape=jax.ShapeDtypeStruct((n1 + 1, w), jnp.float32),
    )(s1p, d1p, s2e, base)

    return out_flat.reshape(n1 + 1, n2 + 1, 3)


# speedup vs baseline: 5.6189x; 5.6189x over previous
import functools

import jax
import jax.numpy as jnp
from jax import lax
from jax.experimental import pallas as pl
from jax.experimental.pallas import tpu as pltpu
from jax.experimental.pallas import tpu_sc as plsc

_VOCAB = 1000
_N1 = 1024
_N2 = 1024
_LANES = 16
_NC = 2
_NS = 16
_NW = _NC * _NS
_PER_W = _N1 // _NW


def _sc_diag_gather_body(seq1_hbm, pw_flat_hbm, out_hbm, idx_raw, idx_sc,
                         vals, sem):
    wid = lax.axis_index("s") * _NC + lax.axis_index("c")
    base = wid * _PER_W
    pltpu.sync_copy(seq1_hbm.at[pl.ds(base, _PER_W)], idx_raw)
    for j in range(_PER_W // _LANES):
        v = idx_raw[pl.ds(j * _LANES, _LANES)]
        idx_sc[pl.ds(j * _LANES, _LANES)] = v * (_VOCAB + 1)
    pltpu.async_copy(pw_flat_hbm.at[idx_sc], vals, sem).wait()
    pltpu.sync_copy(vals, out_hbm.at[pl.ds(base, _PER_W)])


_sc_diag_gather = functools.partial(
    pl.kernel,
    out_type=jax.ShapeDtypeStruct((_N1, 1), jnp.float32),
    mesh=plsc.VectorSubcoreMesh(core_axis_name="c", subcore_axis_name="s"),
    scratch_types=[
        pltpu.VMEM((_PER_W,), jnp.int32),
        pltpu.VMEM((_PER_W,), jnp.int32),
        pltpu.VMEM((_PER_W, 1), jnp.float32),
        pltpu.SemaphoreType.DMA,
    ],
    compiler_params=pltpu.CompilerParams(use_tc_tiling_on_sc=False),
)(_sc_diag_gather_body)


def _fill_body(s1_ref, d1_ref, s2e_ref, base_ref, out_ref):
    eq = s1_ref[...] == s2e_ref[...]
    out_ref[...] = jnp.where(eq, d1_ref[...], base_ref[...])


def kernel(encoded_seq1, encoded_seq2, pw_scores, gap_score):
    n1, n2 = _N1, _N2
    w = 3 * (n2 + 1)
    gap = gap_score.astype(jnp.float32)

    d1 = pw_scores.reshape(-1)[encoded_seq1 * (_VOCAB + 1)].reshape(_N1, 1)

    lane = jnp.arange(w, dtype=jnp.int32)
    ch0 = lane % 3 == 0
    s2rep = jnp.repeat(encoded_seq2, 3, total_repeat_length=3 * n2)
    s2rep = jnp.concatenate(
        [s2rep, jnp.full((3,), -1, dtype=jnp.int32)])
    s2e = jnp.where(ch0, s2rep, -1).reshape(1, w)
    base = jnp.where(ch0, jnp.float32(0.0), gap).reshape(1, w)
    s1p = jnp.concatenate(
        [encoded_seq1, jnp.full((1,), -2, dtype=jnp.int32)]).reshape(n1 + 1, 1)
    d1p = jnp.concatenate(
        [d1, jnp.zeros((1, 1), dtype=jnp.float32)], axis=0)

    bi = 128
    grid = (pl.cdiv(n1 + 1, bi),)
    out_flat = pl.pallas_call(
        _fill_body,
        grid=grid,
        in_specs=[
            pl.BlockSpec((bi, 1), lambda i: (i, 0)),
            pl.BlockSpec((bi, 1), lambda i: (i, 0)),
            pl.BlockSpec((1, w), lambda i: (0, 0)),
            pl.BlockSpec((1, w), lambda i: (0, 0)),
        ],
        out_specs=pl.BlockSpec((bi, w), lambda i: (i, 0)),
        out_shape=jax.ShapeDtypeStruct((n1 + 1, w), jnp.float32),
    )(s1p, d1p, s2e, base)

    return out_flat.reshape(n1 + 1, n2 + 1, 3)


# speedup vs baseline: 22.0885x; 3.9311x over previous
import functools

import jax
import jax.numpy as jnp
from jax import lax
from jax.experimental import pallas as pl
from jax.experimental.pallas import tpu as pltpu
from jax.experimental.pallas import tpu_sc as plsc

_VOCAB = 1000
_N1 = 1024
_N2 = 1024
_LANES = 16
_NC = 2
_NS = 16
_NW = _NC * _NS
_PER_W = _N1 // _NW


def _sc_diag_gather_body(seq1_hbm, pw_flat_hbm, out_hbm, idx_raw, idx_sc,
                         vals, sem):
    wid = lax.axis_index("s") * _NC + lax.axis_index("c")
    base = wid * _PER_W
    pltpu.sync_copy(seq1_hbm.at[pl.ds(base, _PER_W)], idx_raw)
    for j in range(_PER_W // _LANES):
        v = idx_raw[pl.ds(j * _LANES, _LANES)]
        idx_sc[pl.ds(j * _LANES, _LANES)] = v * (_VOCAB + 1)
    pltpu.async_copy(pw_flat_hbm.at[idx_sc], vals, sem).wait()
    pltpu.sync_copy(vals, out_hbm.at[pl.ds(base, _PER_W)])


_sc_diag_gather = functools.partial(
    pl.kernel,
    out_type=jax.ShapeDtypeStruct((_N1, 1), jnp.float32),
    mesh=plsc.VectorSubcoreMesh(core_axis_name="c", subcore_axis_name="s"),
    scratch_types=[
        pltpu.VMEM((_PER_W,), jnp.int32),
        pltpu.VMEM((_PER_W,), jnp.int32),
        pltpu.VMEM((_PER_W, 1), jnp.float32),
        pltpu.SemaphoreType.DMA,
    ],
    compiler_params=pltpu.CompilerParams(use_tc_tiling_on_sc=False),
)(_sc_diag_gather_body)


def _fill_body(s1_ref, d1_ref, s2e_ref, base_ref, out_ref):
    eq = s1_ref[...] == s2e_ref[...]
    out_ref[...] = jnp.where(eq, d1_ref[...], base_ref[...])


def kernel(encoded_seq1, encoded_seq2, pw_scores, gap_score):
    n1, n2 = _N1, _N2
    w = 3 * (n2 + 1)
    gap = gap_score.astype(jnp.float32)

    d1 = pw_scores.reshape(-1)[encoded_seq1 * (_VOCAB + 1)].reshape(_N1, 1)

    lane = jnp.arange(w, dtype=jnp.int32)
    ch0 = lane % 3 == 0
    s2rep = jnp.repeat(encoded_seq2, 3, total_repeat_length=3 * n2)
    s2rep = jnp.concatenate(
        [s2rep, jnp.full((3,), -1, dtype=jnp.int32)])
    s2e = jnp.where(ch0, s2rep, -1).reshape(1, w)
    base = jnp.where(ch0, jnp.float32(0.0), gap).reshape(1, w)
    s1p = jnp.concatenate(
        [encoded_seq1, jnp.full((1,), -2, dtype=jnp.int32)]).reshape(n1 + 1, 1)
    d1p = jnp.concatenate(
        [d1, jnp.zeros((1, 1), dtype=jnp.float32)], axis=0)

    bi = 128
    grid = (pl.cdiv(n1 + 1, bi),)
    out_flat = pl.pallas_call(
        _fill_body,
        grid=grid,
        in_specs=[
            pl.BlockSpec((bi, 1), lambda i: (i, 0)),
            pl.BlockSpec((bi, 1), lambda i: (i, 0)),
            pl.BlockSpec((1, w), lambda i: (0, 0)),
            pl.BlockSpec((1, w), lambda i: (0, 0)),
        ],
        out_specs=pl.BlockSpec((bi, w), lambda i: (i, 0)),
        out_shape=jax.ShapeDtypeStruct((n1 + 1, w), jnp.float32),
    )(s1p, d1p, s2e, base)

    return out_flat


# speedup vs baseline: 37.5614x; 1.7005x over previous
import jax
import jax.numpy as jnp
from jax.experimental import pallas as pl

_VOCAB = 1000
_N1 = 1024
_N2 = 1024


def _fill_body(s1_ref, d1_ref, s2e_ref, gap_ref, out_ref):
    eq = s1_ref[...] == s2e_ref[...]
    gap = gap_ref[0, 0]
    out_ref[0, :, :] = jnp.where(eq, d1_ref[...], jnp.float32(0.0))
    out_ref[1, :, :] = jnp.full_like(out_ref[1, :, :], gap)
    out_ref[2, :, :] = jnp.full_like(out_ref[2, :, :], gap)


def kernel(encoded_seq1, encoded_seq2, pw_scores, gap_score):
    n1, n2 = _N1, _N2
    w = n2 + 1
    gap = gap_score.astype(jnp.float32).reshape(1, 1)

    d1 = pw_scores.reshape(-1)[encoded_seq1 * (_VOCAB + 1)].reshape(n1, 1)

    s2e = jnp.concatenate(
        [encoded_seq2, jnp.full((1,), -1, dtype=jnp.int32)]).reshape(1, w)
    s1p = jnp.concatenate(
        [encoded_seq1, jnp.full((1,), -2, dtype=jnp.int32)]).reshape(n1 + 1, 1)
    d1p = jnp.concatenate(
        [d1, jnp.zeros((1, 1), dtype=jnp.float32)], axis=0)

    bi = 128
    grid = (pl.cdiv(n1 + 1, bi),)
    out3 = pl.pallas_call(
        _fill_body,
        grid=grid,
        in_specs=[
            pl.BlockSpec((bi, 1), lambda i: (i, 0)),
            pl.BlockSpec((bi, 1), lambda i: (i, 0)),
            pl.BlockSpec((1, w), lambda i: (0, 0)),
            pl.BlockSpec((1, 1), lambda i: (0, 0)),
        ],
        out_specs=pl.BlockSpec((3, bi, w), lambda i: (0, i, 0)),
        out_shape=jax.ShapeDtypeStruct((3, n1 + 1, w), jnp.float32),
    )(s1p, d1p, s2e, gap)

    return out3.transpose(1, 2, 0)


# speedup vs baseline: 58.6167x; 1.5606x over previous
import jax
import jax.numpy as jnp
from jax import lax
from jax.experimental import pallas as pl

_VOCAB = 1000
_N1 = 1024
_N2 = 1024
_DB = 128
_NDB = 8


def _diag_body(pw_ref, out_ref):
    i = pl.program_id(0)
    row = lax.broadcasted_iota(jnp.int32, (_DB, _DB), 0)
    lane = lax.broadcasted_iota(jnp.int32, (_DB, _DB), 1)
    sel = (row == lane) & (i * _DB + row < _VOCAB)
    vals = jnp.where(sel, pw_ref[...], jnp.float32(0.0))
    out_ref[...] = jnp.sum(vals, axis=0).reshape(1, 1, _DB)


def _fill_body(s1_ref, diag_ref, s2e_ref, gap_ref, out_ref):
    s1 = s1_ref[...]
    lane = lax.broadcasted_iota(jnp.int32, (1, _DB), 1)
    d1 = jnp.zeros(s1.shape, jnp.float32)
    for b in range(_NDB):
        dr = diag_ref[b, :, :]
        hit = s1 == (lane + b * _DB)
        d1 = d1 + jnp.sum(jnp.where(hit, dr, jnp.float32(0.0)),
                          axis=1, keepdims=True)
    eq = s1 == s2e_ref[...]
    gap = gap_ref[0, 0]
    out_ref[0, :, :] = jnp.where(eq, d1, jnp.float32(0.0))
    out_ref[1, :, :] = jnp.full_like(out_ref[1, :, :], gap)
    out_ref[2, :, :] = jnp.full_like(out_ref[2, :, :], gap)


def kernel(encoded_seq1, encoded_seq2, pw_scores, gap_score):
    n1, n2 = _N1, _N2
    w = n2 + 1
    gap = gap_score.astype(jnp.float32).reshape(1, 1)

    diag = pl.pallas_call(
        _diag_body,
        grid=(_NDB,),
        in_specs=[pl.BlockSpec((_DB, _DB), lambda i: (i, i))],
        out_specs=pl.BlockSpec((1, 1, _DB), lambda i: (i, 0, 0)),
        out_shape=jax.ShapeDtypeStruct((_NDB, 1, _DB), jnp.float32),
    )(pw_scores)

    s2e = jnp.concatenate(
        [encoded_seq2, jnp.full((1,), -1, dtype=jnp.int32)]).reshape(1, w)
    s1p = jnp.concatenate(
        [encoded_seq1, jnp.full((1,), -2, dtype=jnp.int32)]).reshape(n1 + 1, 1)

    bi = 128
    grid = (pl.cdiv(n1 + 1, bi),)
    out3 = pl.pallas_call(
        _fill_body,
        grid=grid,
        in_specs=[
            pl.BlockSpec((bi, 1), lambda i: (i, 0)),
            pl.BlockSpec((_NDB, 1, _DB), lambda i: (0, 0, 0)),
            pl.BlockSpec((1, w), lambda i: (0, 0)),
            pl.BlockSpec((1, 1), lambda i: (0, 0)),
        ],
        out_specs=pl.BlockSpec((3, bi, w), lambda i: (0, i, 0)),
        out_shape=jax.ShapeDtypeStruct((3, n1 + 1, w), jnp.float32),
    )(s1p, diag, s2e, gap)

    return out3.transpose(1, 2, 0)


# speedup vs baseline: 64.9382x; 1.1078x over previous
import jax
import jax.numpy as jnp
from jax import lax
from jax.experimental import pallas as pl

_VOCAB = 1000
_N1 = 1024
_N2 = 1024
_DB = 128
_NDB = 8


def _diag_body(pw_ref, out_ref):
    i = pl.program_id(0)
    row = lax.broadcasted_iota(jnp.int32, (_DB, _DB), 0)
    lane = lax.broadcasted_iota(jnp.int32, (_DB, _DB), 1)
    sel = (row == lane) & (i * _DB + row < _VOCAB)
    vals = jnp.where(sel, pw_ref[...], jnp.float32(0.0))
    out_ref[...] = jnp.sum(vals, axis=0).reshape(1, 1, _DB)


def _fill_body(s1_ref, diag_ref, s2e_ref, gap_ref, out_ref):
    s1 = s1_ref[...]
    lane = lax.broadcasted_iota(jnp.int32, (1, _DB), 1)
    d1 = jnp.zeros(s1.shape, jnp.float32)
    for b in range(_NDB):
        dr = diag_ref[b, :, :]
        hit = s1 == (lane + b * _DB)
        d1 = d1 + jnp.sum(jnp.where(hit, dr, jnp.float32(0.0)),
                          axis=1, keepdims=True)
    eq = s1 == s2e_ref[...]
    gap = gap_ref[0, 0]
    out_ref[0, :, :] = jnp.where(eq, d1, jnp.float32(0.0))
    out_ref[1, :, :] = jnp.full_like(out_ref[1, :, :], gap)
    out_ref[2, :, :] = jnp.full_like(out_ref[2, :, :], gap)


def kernel(encoded_seq1, encoded_seq2, pw_scores, gap_score):
    n1, n2 = _N1, _N2
    w = n2 + 1
    gap = gap_score.astype(jnp.float32).reshape(1, 1)

    diag = pl.pallas_call(
        _diag_body,
        grid=(_NDB,),
        in_specs=[pl.BlockSpec((_DB, _DB), lambda i: (i, i))],
        out_specs=pl.BlockSpec((1, 1, _DB), lambda i: (i, 0, 0)),
        out_shape=jax.ShapeDtypeStruct((_NDB, 1, _DB), jnp.float32),
    )(pw_scores)

    s2e = jnp.concatenate(
        [encoded_seq2, jnp.full((1,), -1, dtype=jnp.int32)]).reshape(1, w)
    s1p = jnp.concatenate(
        [encoded_seq1, jnp.full((1,), -2, dtype=jnp.int32)]).reshape(n1 + 1, 1)

    bi = 256
    grid = (pl.cdiv(n1 + 1, bi),)
    out3 = pl.pallas_call(
        _fill_body,
        grid=grid,
        in_specs=[
            pl.BlockSpec((bi, 1), lambda i: (i, 0)),
            pl.BlockSpec((_NDB, 1, _DB), lambda i: (0, 0, 0)),
            pl.BlockSpec((1, w), lambda i: (0, 0)),
            pl.BlockSpec((1, 1), lambda i: (0, 0)),
        ],
        out_specs=pl.BlockSpec((3, bi, w), lambda i: (0, i, 0)),
        out_shape=jax.ShapeDtypeStruct((3, n1 + 1, w), jnp.float32),
    )(s1p, diag, s2e, gap)

    return out3.transpose(1, 2, 0)


# speedup vs baseline: 65.3151x; 1.0058x over previous
import jax
import jax.numpy as jnp
from jax import lax
from jax.experimental import pallas as pl

_VOCAB = 1000
_N1 = 1024
_N2 = 1024
_DB = 128
_NDB = 8


def _diag_body(pw_ref, out_ref):
    i = pl.program_id(0)
    row = lax.broadcasted_iota(jnp.int32, (_DB, _DB), 0)
    lane = lax.broadcasted_iota(jnp.int32, (_DB, _DB), 1)
    sel = (row == lane) & (i * _DB + row < _VOCAB)
    vals = jnp.where(sel, pw_ref[...], jnp.float32(0.0))
    out_ref[...] = jnp.sum(vals, axis=0).reshape(1, 1, _DB)


def _fill_body(s1_ref, diag_ref, s2e_ref, gap_ref, out_ref):
    s1 = s1_ref[...]
    lane = lax.broadcasted_iota(jnp.int32, (1, _DB), 1)
    d1 = jnp.zeros(s1.shape, jnp.float32)
    for b in range(_NDB):
        dr = diag_ref[b, :, :]
        hit = s1 == (lane + b * _DB)
        d1 = d1 + jnp.sum(jnp.where(hit, dr, jnp.float32(0.0)),
                          axis=1, keepdims=True)
    eq = s1 == s2e_ref[...]
    gap = gap_ref[0, 0]
    out_ref[0, :, :] = jnp.where(eq, d1, jnp.float32(0.0))
    out_ref[1, :, :] = jnp.full_like(out_ref[1, :, :], gap)
    out_ref[2, :, :] = jnp.full_like(out_ref[2, :, :], gap)


def kernel(encoded_seq1, encoded_seq2, pw_scores, gap_score):
    n1, n2 = _N1, _N2
    w = n2 + 1
    gap = gap_score.astype(jnp.float32).reshape(1, 1)

    diag = pl.pallas_call(
        _diag_body,
        grid=(_NDB,),
        in_specs=[pl.BlockSpec((_DB, _DB), lambda i: (i, i))],
        out_specs=pl.BlockSpec((1, 1, _DB), lambda i: (i, 0, 0)),
        out_shape=jax.ShapeDtypeStruct((_NDB, 1, _DB), jnp.float32),
    )(pw_scores)

    s2e = jnp.concatenate(
        [encoded_seq2, jnp.full((1,), -1, dtype=jnp.int32)]).reshape(1, w)
    s1p = jnp.concatenate(
        [encoded_seq1, jnp.full((1,), -2, dtype=jnp.int32)]).reshape(n1 + 1, 1)

    bi = 512
    grid = (pl.cdiv(n1 + 1, bi),)
    out3 = pl.pallas_call(
        _fill_body,
        grid=grid,
        in_specs=[
            pl.BlockSpec((bi, 1), lambda i: (i, 0)),
            pl.BlockSpec((_NDB, 1, _DB), lambda i: (0, 0, 0)),
            pl.BlockSpec((1, w), lambda i: (0, 0)),
            pl.BlockSpec((1, 1), lambda i: (0, 0)),
        ],
        out_specs=pl.BlockSpec((3, bi, w), lambda i: (0, i, 0)),
        out_shape=jax.ShapeDtypeStruct((3, n1 + 1, w), jnp.float32),
    )(s1p, diag, s2e, gap)

    return out3.transpose(1, 2, 0)


# speedup vs baseline: 74.9131x; 1.1469x over previous
import jax
import jax.numpy as jnp
from jax import lax
from jax.experimental import pallas as pl

_VOCAB = 1000
_N1 = 1024
_N2 = 1024
_DB = 256
_NDB = 4


def _diag_body(pw_ref, out_ref):
    i = pl.program_id(0)
    row = lax.broadcasted_iota(jnp.int32, (_DB, _DB), 0)
    lane = lax.broadcasted_iota(jnp.int32, (_DB, _DB), 1)
    sel = (row == lane) & (i * _DB + row < _VOCAB)
    vals = jnp.where(sel, pw_ref[...], jnp.float32(0.0))
    out_ref[...] = jnp.sum(vals, axis=0).reshape(1, 1, _DB)


def _fill_body(s1_ref, diag_ref, s2e_ref, gap_ref, out_ref):
    s1 = s1_ref[...]
    lane = lax.broadcasted_iota(jnp.int32, (1, _DB), 1)
    d1 = jnp.zeros(s1.shape, jnp.float32)
    for b in range(_NDB):
        dr = diag_ref[b, :, :]
        hit = s1 == (lane + b * _DB)
        d1 = d1 + jnp.sum(jnp.where(hit, dr, jnp.float32(0.0)),
                          axis=1, keepdims=True)
    eq = s1 == s2e_ref[...]
    gap = gap_ref[0, 0]
    out_ref[0, :, :] = jnp.where(eq, d1, jnp.float32(0.0))
    out_ref[1, :, :] = jnp.full_like(out_ref[1, :, :], gap)
    out_ref[2, :, :] = jnp.full_like(out_ref[2, :, :], gap)


def kernel(encoded_seq1, encoded_seq2, pw_scores, gap_score):
    n1, n2 = _N1, _N2
    w = n2 + 1
    gap = gap_score.astype(jnp.float32).reshape(1, 1)

    diag = pl.pallas_call(
        _diag_body,
        grid=(_NDB,),
        in_specs=[pl.BlockSpec((_DB, _DB), lambda i: (i, i))],
        out_specs=pl.BlockSpec((1, 1, _DB), lambda i: (i, 0, 0)),
        out_shape=jax.ShapeDtypeStruct((_NDB, 1, _DB), jnp.float32),
    )(pw_scores)

    s2e = jnp.concatenate(
        [encoded_seq2, jnp.full((1,), -1, dtype=jnp.int32)]).reshape(1, w)
    s1p = jnp.concatenate(
        [encoded_seq1, jnp.full((1,), -2, dtype=jnp.int32)]).reshape(n1 + 1, 1)

    bi = 512
    grid = (pl.cdiv(n1 + 1, bi),)
    out3 = pl.pallas_call(
        _fill_body,
        grid=grid,
        in_specs=[
            pl.BlockSpec((bi, 1), lambda i: (i, 0)),
            pl.BlockSpec((_NDB, 1, _DB), lambda i: (0, 0, 0)),
            pl.BlockSpec((1, w), lambda i: (0, 0)),
            pl.BlockSpec((1, 1), lambda i: (0, 0)),
        ],
        out_specs=pl.BlockSpec((3, bi, w), lambda i: (0, i, 0)),
        out_shape=jax.ShapeDtypeStruct((3, n1 + 1, w), jnp.float32),
    )(s1p, diag, s2e, gap)

    return out3.transpose(1, 2, 0)


# speedup vs baseline: 80.7715x; 1.0782x over previous
import jax
import jax.numpy as jnp
from jax import lax
from jax.experimental import pallas as pl

_VOCAB = 1000
_N1 = 1024
_N2 = 1024
_DB = 512
_NDB = 2


def _diag_body(pw_ref, out_ref):
    i = pl.program_id(0)
    row = lax.broadcasted_iota(jnp.int32, (_DB, _DB), 0)
    lane = lax.broadcasted_iota(jnp.int32, (_DB, _DB), 1)
    sel = (row == lane) & (i * _DB + row < _VOCAB)
    vals = jnp.where(sel, pw_ref[...], jnp.float32(0.0))
    out_ref[...] = jnp.sum(vals, axis=0).reshape(1, 1, _DB)


def _fill_body(s1_ref, diag_ref, s2e_ref, gap_ref, out_ref):
    s1 = s1_ref[...]
    lane = lax.broadcasted_iota(jnp.int32, (1, _DB), 1)
    d1 = jnp.zeros(s1.shape, jnp.float32)
    for b in range(_NDB):
        dr = diag_ref[b, :, :]
        hit = s1 == (lane + b * _DB)
        d1 = d1 + jnp.sum(jnp.where(hit, dr, jnp.float32(0.0)),
                          axis=1, keepdims=True)
    eq = s1 == s2e_ref[...]
    gap = gap_ref[0, 0]
    out_ref[0, :, :] = jnp.where(eq, d1, jnp.float32(0.0))
    out_ref[1, :, :] = jnp.full_like(out_ref[1, :, :], gap)
    out_ref[2, :, :] = jnp.full_like(out_ref[2, :, :], gap)


def kernel(encoded_seq1, encoded_seq2, pw_scores, gap_score):
    n1, n2 = _N1, _N2
    w = n2 + 1
    gap = gap_score.astype(jnp.float32).reshape(1, 1)

    diag = pl.pallas_call(
        _diag_body,
        grid=(_NDB,),
        in_specs=[pl.BlockSpec((_DB, _DB), lambda i: (i, i))],
        out_specs=pl.BlockSpec((1, 1, _DB), lambda i: (i, 0, 0)),
        out_shape=jax.ShapeDtypeStruct((_NDB, 1, _DB), jnp.float32),
    )(pw_scores)

    s2e = jnp.concatenate(
        [encoded_seq2, jnp.full((1,), -1, dtype=jnp.int32)]).reshape(1, w)
    s1p = jnp.concatenate(
        [encoded_seq1, jnp.full((1,), -2, dtype=jnp.int32)]).reshape(n1 + 1, 1)

    bi = 512
    grid = (pl.cdiv(n1 + 1, bi),)
    out3 = pl.pallas_call(
        _fill_body,
        grid=grid,
        in_specs=[
            pl.BlockSpec((bi, 1), lambda i: (i, 0)),
            pl.BlockSpec((_NDB, 1, _DB), lambda i: (0, 0, 0)),
            pl.BlockSpec((1, w), lambda i: (0, 0)),
            pl.BlockSpec((1, 1), lambda i: (0, 0)),
        ],
        out_specs=pl.BlockSpec((3, bi, w), lambda i: (0, i, 0)),
        out_shape=jax.ShapeDtypeStruct((3, n1 + 1, w), jnp.float32),
    )(s1p, diag, s2e, gap)

    return out3.transpose(1, 2, 0)


# speedup vs baseline: 100.2889x; 1.2416x over previous
import jax
import jax.numpy as jnp
from jax import lax
from jax.experimental import pallas as pl
from jax.experimental.pallas import tpu as pltpu

_VOCAB = 1000
_N1 = 1024
_N2 = 1024
_DB = 512
_NDB = 2
_BI = 512
_NFILL = 3


def _body(s1_ref, s2_ref, gap_ref, pw_ref, out_ref, diag_scr):
    i = pl.program_id(0)

    @pl.when(i < _NDB)
    def _extract():
        row = lax.broadcasted_iota(jnp.int32, (_DB, _DB), 0)
        lane = lax.broadcasted_iota(jnp.int32, (_DB, _DB), 1)
        sel = (row == lane) & (i * _DB + row < _VOCAB)
        vals = jnp.where(sel, pw_ref[...], jnp.float32(0.0))
        diag_scr[i] = jnp.sum(vals, axis=0).reshape(1, _DB)

    @pl.when(i >= _NDB)
    def _fill():
        s1 = s1_ref[...]
        lane = lax.broadcasted_iota(jnp.int32, (1, _DB), 1)
        d1 = jnp.zeros(s1.shape, jnp.float32)
        for b in range(_NDB):
            dr = diag_scr[b, :, :]
            hit = s1 == (lane + b * _DB)
            d1 = d1 + jnp.sum(jnp.where(hit, dr, jnp.float32(0.0)),
                              axis=1, keepdims=True)
        eq = s1 == s2_ref[...]
        v = jnp.where(eq, d1, jnp.float32(0.0))
        out0 = jnp.concatenate(
            [v, jnp.zeros((_BI, 1), jnp.float32)], axis=1)
        gap = gap_ref[0, 0]
        out_ref[0, :, :] = out0
        out_ref[1, :, :] = jnp.full_like(out_ref[1, :, :], gap)
        out_ref[2, :, :] = jnp.full_like(out_ref[2, :, :], gap)


def kernel(encoded_seq1, encoded_seq2, pw_scores, gap_score):
    n1, n2 = _N1, _N2
    w = n2 + 1
    gap = gap_score.astype(jnp.float32).reshape(1, 1)

    s2r = encoded_seq2.reshape(1, n2)
    s1p = jnp.concatenate(
        [encoded_seq1, jnp.full((1,), -2, dtype=jnp.int32)]).reshape(n1 + 1, 1)

    def _clamp_diag(i):
        m = jnp.minimum(i, _NDB - 1)
        return (m, m)

    def _fill_row(i):
        return (jnp.maximum(i - _NDB, 0), 0)

    out3 = pl.pallas_call(
        _body,
        grid=(_NDB + _NFILL,),
        in_specs=[
            pl.BlockSpec((_BI, 1), _fill_row),
            pl.BlockSpec((1, n2), lambda i: (0, 0)),
            pl.BlockSpec((1, 1), lambda i: (0, 0)),
            pl.BlockSpec((_DB, _DB), _clamp_diag),
        ],
        out_specs=pl.BlockSpec(
            (3, _BI, w), lambda i: (0, jnp.maximum(i - _NDB, 0), 0)),
        out_shape=jax.ShapeDtypeStruct((3, n1 + 1, w), jnp.float32),
        scratch_shapes=[pltpu.VMEM((_NDB, 1, _DB), jnp.float32)],
    )(s1p, s2r, gap, pw_scores)

    return out3.transpose(1, 2, 0)


# speedup vs baseline: 110.5495x; 1.1023x over previous
import jax
import jax.numpy as jnp
from jax import lax
from jax.experimental import pallas as pl
from jax.experimental.pallas import tpu as pltpu

_VOCAB = 1000
_N1 = 1024
_N2 = 1024
_DB = 512
_NDB = 2
_BI = 384
_NFILL = 3


def _body(s1_ref, s2_ref, gap_ref, pw_ref, out_ref, diag_scr):
    i = pl.program_id(0)

    @pl.when(i < _NDB)
    def _extract():
        row = lax.broadcasted_iota(jnp.int32, (_DB, _DB), 0)
        lane = lax.broadcasted_iota(jnp.int32, (_DB, _DB), 1)
        sel = (row == lane) & (i * _DB + row < _VOCAB)
        vals = jnp.where(sel, pw_ref[...], jnp.float32(0.0))
        diag_scr[i] = jnp.sum(vals, axis=0).reshape(1, _DB)

    @pl.when(i >= _NDB)
    def _fill():
        base = jnp.maximum(i - _NDB, 0) * _BI
        s1 = s1_ref[pl.ds(base, _BI)].reshape(_BI, 1)
        lane = lax.broadcasted_iota(jnp.int32, (1, _DB), 1)
        d1 = jnp.zeros(s1.shape, jnp.float32)
        for b in range(_NDB):
            dr = diag_scr[b, :, :]
            hit = s1 == (lane + b * _DB)
            d1 = d1 + jnp.sum(jnp.where(hit, dr, jnp.float32(0.0)),
                              axis=1, keepdims=True)
        s2row = s2_ref[...].reshape(1, _N2)
        eq = s1 == s2row
        v = jnp.where(eq, d1, jnp.float32(0.0))
        out0 = jnp.concatenate(
            [v, jnp.zeros((_BI, 1), jnp.float32)], axis=1)
        gap = gap_ref[0, 0]
        out_ref[0, :, :] = out0
        out_ref[1, :, :] = jnp.full_like(out_ref[1, :, :], gap)
        out_ref[2, :, :] = jnp.full_like(out_ref[2, :, :], gap)


def kernel(encoded_seq1, encoded_seq2, pw_scores, gap_score):
    n1, n2 = _N1, _N2
    w = n2 + 1
    gap = gap_score.astype(jnp.float32).reshape(1, 1)
    s1p = jnp.pad(encoded_seq1, (0, _BI * _NFILL - n1), constant_values=-2)

    def _clamp_diag(i):
        m = jnp.minimum(i, _NDB - 1)
        return (m, m)

    out3 = pl.pallas_call(
        _body,
        grid=(_NDB + _NFILL,),
        in_specs=[
            pl.BlockSpec((_BI * _NFILL,), lambda i: (0,)),
            pl.BlockSpec((_N2,), lambda i: (0,)),
            pl.BlockSpec((1, 1), lambda i: (0, 0)),
            pl.BlockSpec((_DB, _DB), _clamp_diag),
        ],
        out_specs=pl.BlockSpec(
            (3, _BI, w), lambda i: (0, jnp.maximum(i - _NDB, 0), 0)),
        out_shape=jax.ShapeDtypeStruct((3, n1 + 1, w), jnp.float32),
        scratch_shapes=[pltpu.VMEM((_NDB, 1, _DB), jnp.float32)],
    )(s1p, encoded_seq2, gap, pw_scores)

    return out3.transpose(1, 2, 0)
